# baseline (speedup 1.0000x reference)
import functools

import jax
import jax.numpy as jnp
import numpy as np
from jax.experimental import pallas as pl
from jax.experimental.pallas import tpu as pltpu

C = 1024
M = 64
K = 512
H = 8
B = 8
T = 8192
DH = C // H
TT = 512
NT = T // TT
SCALE = float(1.0 / np.sqrt(DH))


def _fused_kernel(bidx_ref, feat_ref, q_ref, wqT_ref, wkT_ref, wvT_ref, b3_ref,
                  owT_ref, ob_ref, lwT_ref, lb_ref, out_ref,
                  qs_ref, numer_ref, denom_ref, cnt_ref, ctx_ref):
    i = pl.program_id(0)

    @pl.when(i == 0)
    def _init():
        q = jnp.dot(q_ref[...], wqT_ref[...],
                    preferred_element_type=jnp.float32) + b3_ref[0:1, :]
        qs_ref[...] = (q * SCALE).astype(jnp.bfloat16)
        numer_ref[...] = jnp.zeros_like(numer_ref)
        denom_ref[...] = jnp.zeros_like(denom_ref)
        cnt_ref[...] = jnp.zeros_like(cnt_ref)

    feat = feat_ref[...]
    k_t = (jnp.dot(feat, wkT_ref[...],
                   preferred_element_type=jnp.float32)
           + b3_ref[1:2, :]).astype(jnp.bfloat16)
    v_t = (jnp.dot(feat, wvT_ref[...],
                   preferred_element_type=jnp.float32)
           + b3_ref[2:3, :]).astype(jnp.bfloat16)

    bidx = bidx_ref[0]
    lane_b = jax.lax.broadcasted_iota(jnp.int32, (TT, B * DH), 1) // DH
    ohx = (bidx == lane_b).astype(jnp.bfloat16)
    oh = (bidx == jax.lax.broadcasted_iota(jnp.int32, (TT, B), 1))
    ohf = oh.astype(jnp.float32)
    cnt_ref[...] += jnp.sum(ohf, axis=0, keepdims=True)

    qs = qs_ref[...]
    for h in range(H):
        k_h = k_t[:, h * DH:(h + 1) * DH]
        v_h = v_t[:, h * DH:(h + 1) * DH]
        s_h = jax.lax.dot_general(qs[:, h * DH:(h + 1) * DH], k_h,
                                  (((1,), (1,)), ((), ())),
                                  preferred_element_type=jnp.float32)
        e_h = jnp.exp(s_h)
        e_bf = e_h.astype(jnp.bfloat16)
        vmask = jnp.concatenate([v_h] * B, axis=1) * ohx
        numer_ref[h] += jnp.dot(e_bf, vmask,
                                preferred_element_type=jnp.float32)
        denom_ref[h] += jnp.dot(e_h, ohf,
                                preferred_element_type=jnp.float32)

    @pl.when(i == NT - 1)
    def _finalize():
        expmat = (jax.lax.broadcasted_iota(jnp.int32, (B, B * DH), 0)
                  == jax.lax.broadcasted_iota(jnp.int32, (B, B * DH), 1) // DH
                  ).astype(jnp.float32)
        for h in range(H):
            d = denom_ref[h]
            inv = 1.0 / jnp.where(d == 0.0, 1.0, d)
            invx = jnp.dot(inv, expmat,
                           preferred_element_type=jnp.float32)
            ctxh = (numer_ref[h] * invx).astype(jnp.bfloat16)
            for b in range(B):
                ctx_ref[b * M:(b + 1) * M, h * DH:(h + 1) * DH] = (
                    ctxh[:, b * DH:(b + 1) * DH])
        attn = jnp.dot(ctx_ref[...], owT_ref[...],
                       preferred_element_type=jnp.float32) + ob_ref[...]
        outr = jnp.dot(attn.astype(jnp.bfloat16), lwT_ref[...],
                       preferred_element_type=jnp.float32) + lb_ref[...]
        attn_d = jnp.dot(b3_ref[2:3, :].astype(jnp.bfloat16), owT_ref[...],
                         preferred_element_type=jnp.float32) + ob_ref[...]
        out_d = jnp.dot(attn_d.astype(jnp.bfloat16), lwT_ref[...],
                        preferred_element_type=jnp.float32) + lb_ref[...]
        alive = (cnt_ref[...] > 0.0).astype(jnp.float32)
        rowsel = (jax.lax.broadcasted_iota(jnp.int32, (B * M, B), 0) // M
                  == jax.lax.broadcasted_iota(jnp.int32, (B * M, B), 1)
                  ).astype(jnp.float32)
        keep = jnp.sum(rowsel * alive, axis=1, keepdims=True)
        out_ref[...] = keep * outr + (1.0 - keep) * out_d


def _run(bidx3, feat, q2, wqT, wkT, wvT, b3, owT, ob2, lwT, lb2):
    return pl.pallas_call(
        _fused_kernel,
        grid=(NT,),
        in_specs=[
            pl.BlockSpec((1, TT, 1), lambda i: (i, 0, 0)),
            pl.BlockSpec((TT, C), lambda i: (i, 0)),
            pl.BlockSpec((M, C), lambda i: (0, 0)),
            pl.BlockSpec((C, C), lambda i: (0, 0)),
            pl.BlockSpec((C, C), lambda i: (0, 0)),
            pl.BlockSpec((C, C), lambda i: (0, 0)),
            pl.BlockSpec((3, C), lambda i: (0, 0)),
            pl.BlockSpec((C, C), lambda i: (0, 0)),
            pl.BlockSpec((1, C), lambda i: (0, 0)),
            pl.BlockSpec((C, K), lambda i: (0, 0)),
            pl.BlockSpec((1, K), lambda i: (0, 0)),
        ],
        out_specs=pl.BlockSpec((B * M, K), lambda i: (0, 0)),
        out_shape=jax.ShapeDtypeStruct((B * M, K), jnp.float32),
        scratch_shapes=[
            pltpu.VMEM((M, C), jnp.bfloat16),
            pltpu.VMEM((H, M, B * DH), jnp.float32),
            pltpu.VMEM((H, M, B), jnp.float32),
            pltpu.VMEM((1, B), jnp.float32),
            pltpu.VMEM((B * M, C), jnp.bfloat16),
        ],
        compiler_params=pltpu.CompilerParams(
            dimension_semantics=("arbitrary",),
        ),
    )(bidx3, feat, q2, wqT, wkT, wvT, b3, owT, ob2, lwT, lb2)


def kernel(feat_all, batch_idx, queries, in_proj_w, in_proj_b, out_w, out_b,
           lin_w, lin_b):
    bidx3 = batch_idx.astype(jnp.int32).reshape(NT, TT, 1)
    q2 = queries.reshape(M, C)
    wqT = in_proj_w[:C].T
    wkT = in_proj_w[C:2 * C].T.astype(jnp.bfloat16)
    wvT = in_proj_w[2 * C:].T.astype(jnp.bfloat16)
    b3 = in_proj_b.reshape(3, C)
    owT = out_w.T.astype(jnp.bfloat16)
    lwT = lin_w.T.astype(jnp.bfloat16)
    ob2 = out_b.reshape(1, C)
    lb2 = lin_b.reshape(1, K)
    out = _run(bidx3, feat_all.astype(jnp.bfloat16), q2, wqT, wkT, wvT, b3,
               owT, ob2, lwT, lb2)
    return out.reshape(B, M, K)


# speedup vs baseline: 1.2350x; 1.2350x over previous
import functools

import jax
import jax.numpy as jnp
import numpy as np
from jax.experimental import pallas as pl
from jax.experimental.pallas import tpu as pltpu

C = 1024
M = 64
K = 512
H = 8
B = 8
T = 8192
DH = C // H
TT = 512
NT = T // TT
SCALE = float(1.0 / np.sqrt(DH))


def _fused_kernel(bidx_ref, feat_ref, q_ref, wqT_ref, wkT_ref, wvT_ref, b3_ref,
                  owT_ref, ob_ref, lwT_ref, lb_ref, out_ref,
                  qs_ref, numer_ref, denom_ref, cnt_ref):
    i = pl.program_id(0)

    @pl.when(i == 0)
    def _init():
        q = jnp.dot(q_ref[...], wqT_ref[...],
                    preferred_element_type=jnp.float32) + b3_ref[0:1, :]
        qs_ref[...] = q * SCALE
        numer_ref[...] = jnp.zeros_like(numer_ref)
        denom_ref[...] = jnp.zeros_like(denom_ref)
        cnt_ref[...] = jnp.zeros_like(cnt_ref)

    feat = feat_ref[...]
    k_t = jnp.dot(feat, wkT_ref[...],
                  preferred_element_type=jnp.float32) + b3_ref[1:2, :]
    v_t = jnp.dot(feat, wvT_ref[...],
                  preferred_element_type=jnp.float32) + b3_ref[2:3, :]

    bidx = bidx_ref[0]
    row_b = jax.lax.broadcasted_iota(jnp.int32, (B * M, TT), 0) // M
    maskE = (row_b == bidx).astype(jnp.float32)
    cnt_new = cnt_ref[...] + jnp.sum(maskE, axis=1, keepdims=True)

    qs = qs_ref[...]
    nparts = []
    dparts = []
    for h in range(H):
        k_h = k_t[:, h * DH:(h + 1) * DH]
        v_h = v_t[:, h * DH:(h + 1) * DH]
        s_h = jax.lax.dot_general(qs[:, h * DH:(h + 1) * DH], k_h,
                                  (((1,), (1,)), ((), ())),
                                  preferred_element_type=jnp.float32)
        e_h = jnp.exp(s_h)
        e_tiled = jnp.concatenate([e_h] * B, axis=0)
        E = e_tiled * maskE
        nparts.append(jnp.dot(E, v_h,
                              preferred_element_type=jnp.float32))
        dparts.append(jnp.sum(E, axis=1, keepdims=True))
    numer_ref[...] += jnp.concatenate(nparts, axis=1)
    denom_ref[...] += jnp.concatenate(dparts, axis=1)
    cnt_ref[...] = cnt_new

    @pl.when(i == NT - 1)
    def _finalize():
        expmat = (jax.lax.broadcasted_iota(jnp.int32, (H, C), 0)
                  == jax.lax.broadcasted_iota(jnp.int32, (H, C), 1) // DH
                  ).astype(jnp.float32)
        d = denom_ref[...]
        inv = 1.0 / jnp.where(d == 0.0, 1.0, d)
        invx = jnp.dot(inv, expmat,
                       preferred_element_type=jnp.float32)
        ctx = numer_ref[...] * invx
        attn = jnp.dot(ctx, owT_ref[...],
                       preferred_element_type=jnp.float32) + ob_ref[...]
        outr = jnp.dot(attn, lwT_ref[...],
                       preferred_element_type=jnp.float32) + lb_ref[...]
        attn_d = jnp.dot(b3_ref[2:3, :], owT_ref[...],
                         preferred_element_type=jnp.float32) + ob_ref[...]
        out_d = jnp.dot(attn_d, lwT_ref[...],
                        preferred_element_type=jnp.float32) + lb_ref[...]
        keep = (cnt_ref[...] > 0.0).astype(jnp.float32)
        out_ref[...] = keep * outr + (1.0 - keep) * out_d


def _run(bidx3, feat, q2, wqT, wkT, wvT, b3, owT, ob2, lwT, lb2):
    return pl.pallas_call(
        _fused_kernel,
        grid=(NT,),
        in_specs=[
            pl.BlockSpec((1, 1, TT), lambda i: (i, 0, 0)),
            pl.BlockSpec((TT, C), lambda i: (i, 0)),
            pl.BlockSpec((M, C), lambda i: (0, 0)),
            pl.BlockSpec((C, C), lambda i: (0, 0)),
            pl.BlockSpec((C, C), lambda i: (0, 0)),
            pl.BlockSpec((C, C), lambda i: (0, 0)),
            pl.BlockSpec((3, C), lambda i: (0, 0)),
            pl.BlockSpec((C, C), lambda i: (0, 0)),
            pl.BlockSpec((1, C), lambda i: (0, 0)),
            pl.BlockSpec((C, K), lambda i: (0, 0)),
            pl.BlockSpec((1, K), lambda i: (0, 0)),
        ],
        out_specs=pl.BlockSpec((B * M, K), lambda i: (0, 0)),
        out_shape=jax.ShapeDtypeStruct((B * M, K), jnp.float32),
        scratch_shapes=[
            pltpu.VMEM((M, C), jnp.float32),
            pltpu.VMEM((B * M, C), jnp.float32),
            pltpu.VMEM((B * M, H), jnp.float32),
            pltpu.VMEM((B * M, 1), jnp.float32),
        ],
        compiler_params=pltpu.CompilerParams(
            dimension_semantics=("arbitrary",),
        ),
    )(bidx3, feat, q2, wqT, wkT, wvT, b3, owT, ob2, lwT, lb2)


def kernel(feat_all, batch_idx, queries, in_proj_w, in_proj_b, out_w, out_b,
           lin_w, lin_b):
    bidx3 = batch_idx.astype(jnp.int32).reshape(NT, 1, TT)
    q2 = queries.reshape(M, C)
    wqT = in_proj_w[:C].T
    wkT = in_proj_w[C:2 * C].T
    wvT = in_proj_w[2 * C:].T
    b3 = in_proj_b.reshape(3, C)
    owT = out_w.T
    lwT = lin_w.T
    ob2 = out_b.reshape(1, C)
    lb2 = lin_b.reshape(1, K)
    out = _run(bidx3, feat_all, q2, wqT, wkT, wvT, b3, owT, ob2, lwT, lb2)
    return out.reshape(B, M, K)


# speedup vs baseline: 1.4656x; 1.1868x over previous
import functools

import jax
import jax.numpy as jnp
import numpy as np
from jax.experimental import pallas as pl
from jax.experimental.pallas import tpu as pltpu

C = 1024
M = 64
K = 512
H = 8
B = 8
T = 8192
DH = C // H
TT = 512
NT = T // TT
SCALE = float(1.0 / np.sqrt(DH))


def _fused_kernel(bidx_ref, feat_ref, q_ref, wqT_ref, wkT_ref, wvT_ref, b3_ref,
                  owT_ref, ob_ref, lwT_ref, lb_ref, out_ref,
                  qs_ref, numer_ref, denom_ref, cnt_ref):
    i = pl.program_id(0)

    @pl.when(i == 0)
    def _init():
        q = jax.lax.dot_general(q_ref[...], wqT_ref[...],
                                (((1,), (1,)), ((), ())),
                                preferred_element_type=jnp.float32) + b3_ref[0:1, :]
        qs_ref[...] = q * SCALE
        numer_ref[...] = jnp.zeros_like(numer_ref)
        denom_ref[...] = jnp.zeros_like(denom_ref)
        cnt_ref[...] = jnp.zeros_like(cnt_ref)

    feat = feat_ref[...]
    k_t = jax.lax.dot_general(feat, wkT_ref[...], (((1,), (1,)), ((), ())),
                              preferred_element_type=jnp.float32) + b3_ref[1:2, :]
    v_t = jax.lax.dot_general(feat, wvT_ref[...], (((1,), (1,)), ((), ())),
                              preferred_element_type=jnp.float32) + b3_ref[2:3, :]

    bidx = bidx_ref[0]
    row_b = jax.lax.broadcasted_iota(jnp.int32, (B * M, TT), 0) // M
    maskE = (row_b == bidx).astype(jnp.float32)
    cnt_new = cnt_ref[...] + jnp.sum(maskE, axis=1, keepdims=True)

    qs = qs_ref[...]
    nparts = []
    dparts = []
    for h in range(H):
        k_h = k_t[:, h * DH:(h + 1) * DH]
        v_h = v_t[:, h * DH:(h + 1) * DH]
        s_h = jax.lax.dot_general(qs[:, h * DH:(h + 1) * DH], k_h,
                                  (((1,), (1,)), ((), ())),
                                  preferred_element_type=jnp.float32)
        e_h = jnp.exp(s_h)
        e_tiled = jnp.concatenate([e_h] * B, axis=0)
        E = e_tiled * maskE
        nparts.append(jnp.dot(E, v_h,
                              preferred_element_type=jnp.float32))
        dparts.append(jnp.sum(E, axis=1, keepdims=True))
    numer_ref[...] += jnp.concatenate(nparts, axis=1)
    denom_ref[...] += jnp.concatenate(dparts, axis=1)
    cnt_ref[...] = cnt_new

    @pl.when(i == NT - 1)
    def _finalize():
        expmat = (jax.lax.broadcasted_iota(jnp.int32, (H, C), 0)
                  == jax.lax.broadcasted_iota(jnp.int32, (H, C), 1) // DH
                  ).astype(jnp.float32)
        d = denom_ref[...]
        inv = 1.0 / jnp.where(d == 0.0, 1.0, d)
        invx = jnp.dot(inv, expmat,
                       preferred_element_type=jnp.float32)
        ctx = numer_ref[...] * invx
        attn = jax.lax.dot_general(ctx, owT_ref[...], (((1,), (1,)), ((), ())),
                                   preferred_element_type=jnp.float32) + ob_ref[...]
        outr = jax.lax.dot_general(attn, lwT_ref[...], (((1,), (1,)), ((), ())),
                                   preferred_element_type=jnp.float32) + lb_ref[...]
        attn_d = jax.lax.dot_general(b3_ref[2:3, :], owT_ref[...],
                                     (((1,), (1,)), ((), ())),
                                     preferred_element_type=jnp.float32) + ob_ref[...]
        out_d = jax.lax.dot_general(attn_d, lwT_ref[...],
                                    (((1,), (1,)), ((), ())),
                                    preferred_element_type=jnp.float32) + lb_ref[...]
        keep = (cnt_ref[...] > 0.0).astype(jnp.float32)
        out_ref[...] = keep * outr + (1.0 - keep) * out_d


def _run(bidx3, feat, q2, wqT, wkT, wvT, b3, owT, ob2, lwT, lb2):
    return pl.pallas_call(
        _fused_kernel,
        grid=(NT,),
        in_specs=[
            pl.BlockSpec((1, 1, TT), lambda i: (i, 0, 0)),
            pl.BlockSpec((TT, C), lambda i: (i, 0)),
            pl.BlockSpec((M, C), lambda i: (0, 0)),
            pl.BlockSpec((C, C), lambda i: (0, 0)),
            pl.BlockSpec((C, C), lambda i: (0, 0)),
            pl.BlockSpec((C, C), lambda i: (0, 0)),
            pl.BlockSpec((3, C), lambda i: (0, 0)),
            pl.BlockSpec((C, C), lambda i: (0, 0)),
            pl.BlockSpec((1, C), lambda i: (0, 0)),
            pl.BlockSpec((K, C), lambda i: (0, 0)),
            pl.BlockSpec((1, K), lambda i: (0, 0)),
        ],
        out_specs=pl.BlockSpec((B * M, K), lambda i: (0, 0)),
        out_shape=jax.ShapeDtypeStruct((B * M, K), jnp.float32),
        scratch_shapes=[
            pltpu.VMEM((M, C), jnp.float32),
            pltpu.VMEM((B * M, C), jnp.float32),
            pltpu.VMEM((B * M, H), jnp.float32),
            pltpu.VMEM((B * M, 1), jnp.float32),
        ],
        compiler_params=pltpu.CompilerParams(
            dimension_semantics=("arbitrary",),
        ),
    )(bidx3, feat, q2, wqT, wkT, wvT, b3, owT, ob2, lwT, lb2)


def kernel(feat_all, batch_idx, queries, in_proj_w, in_proj_b, out_w, out_b,
           lin_w, lin_b):
    bidx3 = batch_idx.astype(jnp.int32).reshape(NT, 1, TT)
    q2 = queries.reshape(M, C)
    wqT = in_proj_w[:C]
    wkT = in_proj_w[C:2 * C]
    wvT = in_proj_w[2 * C:]
    b3 = in_proj_b.reshape(3, C)
    owT = out_w
    lwT = lin_w
    ob2 = out_b.reshape(1, C)
    lb2 = lin_b.reshape(1, K)
    out = _run(bidx3, feat_all, q2, wqT, wkT, wvT, b3, owT, ob2, lwT, lb2)
    return out.reshape(B, M, K)


# speedup vs baseline: 1.6340x; 1.1149x over previous
import functools

import jax
import jax.numpy as jnp
import numpy as np
from jax.experimental import pallas as pl
from jax.experimental.pallas import tpu as pltpu

C = 1024
M = 64
K = 512
H = 8
B = 8
T = 8192
DH = C // H
TT = 512
NT = T // TT
SCALE = float(1.0 / np.sqrt(DH))


def _fused_kernel(bidx_ref, feat_ref, q_ref, ipw_ref, b3_ref,
                  owT_ref, ob_ref, lwT_ref, lb_ref, out_ref,
                  qs_ref, numer_ref, denom_ref, cnt_ref):
    i = pl.program_id(0)

    @pl.when(i == 0)
    def _init():
        q = jax.lax.dot_general(q_ref[...], ipw_ref[0:C, :],
                                (((1,), (1,)), ((), ())),
                                preferred_element_type=jnp.float32) + b3_ref[0:1, :]
        qs_ref[...] = q * SCALE
        numer_ref[...] = jnp.zeros_like(numer_ref)
        denom_ref[...] = jnp.zeros_like(denom_ref)
        cnt_ref[...] = jnp.zeros_like(cnt_ref)

    feat = feat_ref[...]
    k_t = jax.lax.dot_general(feat, ipw_ref[C:2 * C, :],
                              (((1,), (1,)), ((), ())),
                              preferred_element_type=jnp.float32) + b3_ref[1:2, :]
    v_t = jax.lax.dot_general(feat, ipw_ref[2 * C:3 * C, :],
                              (((1,), (1,)), ((), ())),
                              preferred_element_type=jnp.float32) + b3_ref[2:3, :]

    bidx = bidx_ref[0]
    row_b = jax.lax.broadcasted_iota(jnp.int32, (B * M, TT), 0) // M
    maskE = (row_b == bidx).astype(jnp.float32)
    oh_bt = (jax.lax.broadcasted_iota(jnp.int32, (B, TT), 0)
             == bidx).astype(jnp.float32)
    cnt_new = cnt_ref[...] + oh_bt

    qs = qs_ref[...]
    nparts = []
    dparts = []
    for h in range(H):
        k_h = k_t[:, h * DH:(h + 1) * DH]
        v_h = v_t[:, h * DH:(h + 1) * DH]
        s_h = jax.lax.dot_general(qs[:, h * DH:(h + 1) * DH], k_h,
                                  (((1,), (1,)), ((), ())),
                                  preferred_element_type=jnp.float32)
        e_h = jnp.exp(s_h)
        e_tiled = jnp.concatenate([e_h] * B, axis=0)
        E = e_tiled * maskE
        nparts.append(jnp.dot(E, v_h,
                              preferred_element_type=jnp.float32))
        dparts.append(jnp.sum(E, axis=1, keepdims=True))
    numer_ref[...] += jnp.concatenate(nparts, axis=1)
    denom_ref[...] += jnp.concatenate(dparts, axis=1)
    cnt_ref[...] = cnt_new

    @pl.when(i == NT - 1)
    def _finalize():
        expmat = (jax.lax.broadcasted_iota(jnp.int32, (H, C), 0)
                  == jax.lax.broadcasted_iota(jnp.int32, (H, C), 1) // DH
                  ).astype(jnp.float32)
        d = denom_ref[...]
        inv = 1.0 / jnp.where(d == 0.0, 1.0, d)
        invx = jnp.dot(inv, expmat,
                       preferred_element_type=jnp.float32)
        ctx = numer_ref[...] * invx
        attn = jax.lax.dot_general(ctx, owT_ref[...], (((1,), (1,)), ((), ())),
                                   preferred_element_type=jnp.float32) + ob_ref[...]
        outr = jax.lax.dot_general(attn, lwT_ref[...], (((1,), (1,)), ((), ())),
                                   preferred_element_type=jnp.float32) + lb_ref[...]
        attn_d = jax.lax.dot_general(b3_ref[2:3, :], owT_ref[...],
                                     (((1,), (1,)), ((), ())),
                                     preferred_element_type=jnp.float32) + ob_ref[...]
        out_d = jax.lax.dot_general(attn_d, lwT_ref[...],
                                    (((1,), (1,)), ((), ())),
                                    preferred_element_type=jnp.float32) + lb_ref[...]
        alive = (jnp.sum(cnt_ref[...], axis=1, keepdims=True)
                 > 0.0).astype(jnp.float32)
        rowsel = (jax.lax.broadcasted_iota(jnp.int32, (B * M, B), 0) // M
                  == jax.lax.broadcasted_iota(jnp.int32, (B * M, B), 1)
                  ).astype(jnp.float32)
        keep = jnp.dot(rowsel, alive,
                       preferred_element_type=jnp.float32)
        out_ref[...] = keep * outr + (1.0 - keep) * out_d


def _run(bidx3, feat, q2, ipw, b3, owT, ob2, lwT, lb2):
    return pl.pallas_call(
        _fused_kernel,
        grid=(NT,),
        in_specs=[
            pl.BlockSpec((1, 1, TT), lambda i: (i, 0, 0)),
            pl.BlockSpec((TT, C), lambda i: (i, 0)),
            pl.BlockSpec((M, C), lambda i: (0, 0)),
            pl.BlockSpec((3 * C, C), lambda i: (0, 0)),
            pl.BlockSpec((3, C), lambda i: (0, 0)),
            pl.BlockSpec((C, C), lambda i: (0, 0)),
            pl.BlockSpec((1, C), lambda i: (0, 0)),
            pl.BlockSpec((K, C), lambda i: (0, 0)),
            pl.BlockSpec((1, K), lambda i: (0, 0)),
        ],
        out_specs=pl.BlockSpec((B * M, K), lambda i: (0, 0)),
        out_shape=jax.ShapeDtypeStruct((B * M, K), jnp.float32),
        scratch_shapes=[
            pltpu.VMEM((M, C), jnp.float32),
            pltpu.VMEM((B * M, C), jnp.float32),
            pltpu.VMEM((B * M, H), jnp.float32),
            pltpu.VMEM((B, TT), jnp.float32),
        ],
        compiler_params=pltpu.CompilerParams(
            dimension_semantics=("arbitrary",),
        ),
    )(bidx3, feat, q2, ipw, b3, owT, ob2, lwT, lb2)


def kernel(feat_all, batch_idx, queries, in_proj_w, in_proj_b, out_w, out_b,
           lin_w, lin_b):
    bidx3 = batch_idx.astype(jnp.int32).reshape(NT, 1, TT)
    q2 = queries.reshape(M, C)
    b3 = in_proj_b.reshape(3, C)
    ob2 = out_b.reshape(1, C)
    lb2 = lin_b.reshape(1, K)
    out = _run(bidx3, feat_all, q2, in_proj_w, b3, out_w, ob2, lin_w, lb2)
    return out.reshape(B, M, K)
